# Initial kernel scaffold; baseline (speedup 1.0000x reference)
import jax
import jax.numpy as jnp
from jax.experimental import pallas as pl


def kernel(word_ids, context_ids, table, fc_w, fc_b):
    raise NotImplementedError("write your pallas kernel here")


# speedup vs baseline: 2.7550x; 2.7550x over previous
import functools

import jax
import jax.numpy as jnp
from jax import lax
from jax.experimental import pallas as pl
from jax.experimental.pallas import tpu as pltpu
from jax.experimental.pallas import tpu_sc as plsc

B = 16384
L = 50
EMB_DIM = 64
TOTAL = B * L
NW = 32
PER_W = TOTAL // NW
CHUNK = 512
IDXW = 128
ROWS_PER_CHUNK = CHUNK // IDXW
N_CHUNKS = PER_W // CHUNK
GROUPS = CHUNK // 16
UNROLL = 4


def _sc_body(wi_hbm, ci_hbm, table_hbm, params_hbm, out_hbm,
             wi_v, ci_v, wr, cr, outb, pv, sem):
    nc = 2
    wid = lax.axis_index("s") * nc + lax.axis_index("c")

    pltpu.sync_copy(params_hbm, pv)
    bias = pv[pl.ds(EMB_DIM, 16)]

    idx_row0 = wid * (PER_W // IDXW)

    fw = [pv[pl.ds(k * 16, 16)] for k in range(EMB_DIM // 16)]
    mask15 = lax.iota(jnp.int32, 16) == 15

    def chunk_body(n, _):
        row = idx_row0 + n * ROWS_PER_CHUNK
        pltpu.sync_copy(wi_hbm.at[pl.ds(row, ROWS_PER_CHUNK)], wi_v)
        pltpu.sync_copy(ci_hbm.at[pl.ds(row, ROWS_PER_CHUNK)], ci_v)
        cps = []
        for j in range(ROWS_PER_CHUNK):
            cps.append(pltpu.async_copy(
                table_hbm.at[wi_v.at[j]], wr.at[pl.ds(j * IDXW, IDXW)], sem))
            cps.append(pltpu.async_copy(
                table_hbm.at[ci_v.at[j]], cr.at[pl.ds(j * IDXW, IDXW)], sem))
        for cp in cps:
            cp.wait()

        def look_body(i, _):
            for u in range(UNROLL):
                ii = i * UNROLL + u
                acc = None
                for k in range(EMB_DIM // 16):
                    t = wr[ii, pl.ds(k * 16, 16)] * cr[ii, pl.ds(k * 16, 16)]
                    t = t * fw[k]
                    acc = t if acc is None else acc + t
                cum = plsc.cumsum(acc)
                plsc.store_compressed(outb.at[pl.ds(ii, 16)], cum, mask=mask15)
            return 0

        lax.fori_loop(0, CHUNK // UNROLL, look_body, 0)

        def sig_body(g, _):
            v = outb[pl.ds(g * 16, 16)] + bias
            outb[pl.ds(g * 16, 16)] = 1.0 / (1.0 + jnp.exp(-v))
            return 0

        lax.fori_loop(0, GROUPS, sig_body, 0)
        pltpu.sync_copy(outb.at[pl.ds(0, CHUNK)],
                        out_hbm.at[pl.ds(wid * PER_W + n * CHUNK, CHUNK)])
        return 0

    lax.fori_loop(0, N_CHUNKS, chunk_body, 0)


@jax.jit
def _run(wi2d, ci2d, table, params):
    mesh = plsc.VectorSubcoreMesh(core_axis_name="c", subcore_axis_name="s")
    kern = pl.kernel(
        _sc_body,
        out_type=jax.ShapeDtypeStruct((TOTAL,), jnp.float32),
        mesh=mesh,
        scratch_types=[
            pltpu.VMEM((ROWS_PER_CHUNK, IDXW), jnp.int32),
            pltpu.VMEM((ROWS_PER_CHUNK, IDXW), jnp.int32),
            pltpu.VMEM((CHUNK, EMB_DIM), jnp.float32),
            pltpu.VMEM((CHUNK, EMB_DIM), jnp.float32),
            pltpu.VMEM((CHUNK + 16,), jnp.float32),
            pltpu.VMEM((EMB_DIM + 16,), jnp.float32),
            pltpu.SemaphoreType.DMA,
        ],
        compiler_params=pltpu.CompilerParams(
            needs_layout_passes=False, use_tc_tiling_on_sc=False),
    )
    return kern(wi2d, ci2d, table, params)


def kernel(word_ids, context_ids, table, fc_w, fc_b):
    wi2d = word_ids.reshape(TOTAL // IDXW, IDXW).astype(jnp.int32)
    ci2d = context_ids.reshape(TOTAL // IDXW, IDXW).astype(jnp.int32)
    params = jnp.concatenate(
        [fc_w.reshape(EMB_DIM), jnp.broadcast_to(fc_b, (16,))]).astype(jnp.float32)
    out = _run(wi2d, ci2d, table.astype(jnp.float32), params)
    return out.reshape(B, L, 1)


# speedup vs baseline: 2.9773x; 1.0807x over previous
import jax
import jax.numpy as jnp
from jax import lax
from jax.experimental import pallas as pl
from jax.experimental.pallas import tpu as pltpu
from jax.experimental.pallas import tpu_sc as plsc

B = 16384
L = 50
EMB_DIM = 64
TOTAL = B * L
NW = 32
PER_W = TOTAL // NW
CHUNK = 256
IDXW = 128
ROWS_PER_CHUNK = CHUNK // IDXW
N_CHUNKS = PER_W // CHUNK
GROUPS = CHUNK // 16
UNROLL = 8


def _sc_body(wi_hbm, ci_hbm, table_hbm, params_hbm, out_hbm,
             wi_v, ci_v, wr, cr, outb, pv, sems):
    nc = 2
    wid = lax.axis_index("s") * nc + lax.axis_index("c")

    pltpu.sync_copy(params_hbm, pv)
    bias = pv[pl.ds(EMB_DIM, 16)]
    fw = [pv[pl.ds(k * 16, 16)] for k in range(EMB_DIM // 16)]
    mask15 = lax.iota(jnp.int32, 16) == 15

    idx_row0 = wid * (PER_W // IDXW)
    out_base = wid * PER_W

    def start_gathers(c, buf):
        row = idx_row0 + c * ROWS_PER_CHUNK
        pltpu.sync_copy(wi_hbm.at[pl.ds(row, ROWS_PER_CHUNK)], wi_v.at[buf])
        pltpu.sync_copy(ci_hbm.at[pl.ds(row, ROWS_PER_CHUNK)], ci_v.at[buf])
        for j in range(ROWS_PER_CHUNK):
            pltpu.async_copy(table_hbm.at[wi_v.at[buf, j]],
                             wr.at[buf, pl.ds(j * IDXW, IDXW)], sems.at[buf])
            pltpu.async_copy(table_hbm.at[ci_v.at[buf, j]],
                             cr.at[buf, pl.ds(j * IDXW, IDXW)], sems.at[buf])

    def wait_gathers(buf):
        for j in range(ROWS_PER_CHUNK):
            pltpu.make_async_copy(table_hbm.at[wi_v.at[buf, j]],
                                  wr.at[buf, pl.ds(j * IDXW, IDXW)],
                                  sems.at[buf]).wait()
            pltpu.make_async_copy(table_hbm.at[ci_v.at[buf, j]],
                                  cr.at[buf, pl.ds(j * IDXW, IDXW)],
                                  sems.at[buf]).wait()

    def compute_chunk(c, buf):
        def look_body(i, _):
            for u in range(UNROLL):
                ii = i * UNROLL + u
                acc = None
                for k in range(EMB_DIM // 16):
                    t = (wr[buf, ii, pl.ds(k * 16, 16)]
                         * cr[buf, ii, pl.ds(k * 16, 16)]) * fw[k]
                    acc = t if acc is None else acc + t
                cum = plsc.cumsum(acc)
                plsc.store_compressed(outb.at[pl.ds(ii, 16)], cum, mask=mask15)
            return 0

        lax.fori_loop(0, CHUNK // UNROLL, look_body, 0)

        def sig_body(g, _):
            v = outb[pl.ds(g * 16, 16)] + bias
            outb[pl.ds(g * 16, 16)] = 1.0 / (1.0 + jnp.exp(-v))
            return 0

        lax.fori_loop(0, GROUPS, sig_body, 0)
        pltpu.sync_copy(outb.at[pl.ds(0, CHUNK)],
                        out_hbm.at[pl.ds(out_base + c * CHUNK, CHUNK)])

    start_gathers(0, 0)

    def pair_body(m, _):
        c0 = m * 2
        start_gathers(c0 + 1, 1)
        wait_gathers(0)
        compute_chunk(c0, 0)

        @pl.when(c0 + 2 < N_CHUNKS)
        def _():
            start_gathers(c0 + 2, 0)

        wait_gathers(1)
        compute_chunk(c0 + 1, 1)
        return 0

    lax.fori_loop(0, N_CHUNKS // 2, pair_body, 0)


@jax.jit
def _run(wi2d, ci2d, table, params):
    mesh = plsc.VectorSubcoreMesh(core_axis_name="c", subcore_axis_name="s")
    kern = pl.kernel(
        _sc_body,
        out_type=jax.ShapeDtypeStruct((TOTAL,), jnp.float32),
        mesh=mesh,
        scratch_types=[
            pltpu.VMEM((2, ROWS_PER_CHUNK, IDXW), jnp.int32),
            pltpu.VMEM((2, ROWS_PER_CHUNK, IDXW), jnp.int32),
            pltpu.VMEM((2, CHUNK, EMB_DIM), jnp.float32),
            pltpu.VMEM((2, CHUNK, EMB_DIM), jnp.float32),
            pltpu.VMEM((CHUNK + 16,), jnp.float32),
            pltpu.VMEM((EMB_DIM + 16,), jnp.float32),
            pltpu.SemaphoreType.DMA((2,)),
        ],
        compiler_params=pltpu.CompilerParams(
            needs_layout_passes=False, use_tc_tiling_on_sc=False),
    )
    return kern(wi2d, ci2d, table, params)


def kernel(word_ids, context_ids, table, fc_w, fc_b):
    wi2d = word_ids.reshape(TOTAL // IDXW, IDXW).astype(jnp.int32)
    ci2d = context_ids.reshape(TOTAL // IDXW, IDXW).astype(jnp.int32)
    params = jnp.concatenate(
        [fc_w.reshape(EMB_DIM), jnp.broadcast_to(fc_b, (16,))]).astype(jnp.float32)
    out = _run(wi2d, ci2d, table.astype(jnp.float32), params)
    return out.reshape(B, L, 1)


# speedup vs baseline: 3.8471x; 1.2921x over previous
import jax
import jax.numpy as jnp
from jax import lax
from jax.experimental import pallas as pl
from jax.experimental.pallas import tpu as pltpu
from jax.experimental.pallas import tpu_sc as plsc

B = 16384
L = 50
EMB_DIM = 64
TOTAL = B * L
NW = 32
PER_W = TOTAL // NW
CHUNK = 256
IDXW = 128
ROWS_PER_CHUNK = CHUNK // IDXW
N_CHUNKS = PER_W // CHUNK
GROUPS = CHUNK // 16
UNROLL = 8


def _sc_body(wi_hbm, ci_hbm, table_hbm, params_hbm, out_hbm,
             wi_v, ci_v, wr, cr, outb, pv, sems):
    nc = 2
    wid = lax.axis_index("s") * nc + lax.axis_index("c")

    pltpu.sync_copy(params_hbm, pv)
    bias = pv[pl.ds(EMB_DIM, 16)]
    fw = [pv[pl.ds(k * 16, 16)] for k in range(EMB_DIM // 16)]
    mask15 = lax.iota(jnp.int32, 16) == 15

    idx_row0 = wid * (PER_W // IDXW)
    out_base = wid * PER_W

    def start_gathers(c, buf):
        row = idx_row0 + c * ROWS_PER_CHUNK
        pltpu.sync_copy(wi_hbm.at[pl.ds(row, ROWS_PER_CHUNK)], wi_v.at[buf])
        pltpu.sync_copy(ci_hbm.at[pl.ds(row, ROWS_PER_CHUNK)], ci_v.at[buf])
        for j in range(ROWS_PER_CHUNK):
            pltpu.async_copy(table_hbm.at[wi_v.at[buf, j]],
                             wr.at[buf, pl.ds(j * IDXW, IDXW)], sems.at[buf])
            pltpu.async_copy(table_hbm.at[ci_v.at[buf, j]],
                             cr.at[buf, pl.ds(j * IDXW, IDXW)], sems.at[buf])

    def wait_gathers(buf):
        for j in range(ROWS_PER_CHUNK):
            pltpu.make_async_copy(table_hbm.at[wi_v.at[buf, j]],
                                  wr.at[buf, pl.ds(j * IDXW, IDXW)],
                                  sems.at[buf]).wait()
            pltpu.make_async_copy(table_hbm.at[ci_v.at[buf, j]],
                                  cr.at[buf, pl.ds(j * IDXW, IDXW)],
                                  sems.at[buf]).wait()

    def compute_chunk(c, buf):
        def look_body(i, _):
            accs = []
            for u in range(UNROLL):
                ii = i * UNROLL + u
                acc = None
                for k in range(EMB_DIM // 16):
                    t = (wr[buf, ii, pl.ds(k * 16, 16)]
                         * cr[buf, ii, pl.ds(k * 16, 16)]) * fw[k]
                    acc = t if acc is None else acc + t
                accs.append(acc)
            cums = [plsc.cumsum(a) for a in accs]
            for u, cum in enumerate(cums):
                plsc.store_compressed(outb.at[pl.ds(i * UNROLL + u, 16)],
                                      cum, mask=mask15)
            return 0

        lax.fori_loop(0, CHUNK // UNROLL, look_body, 0)

        def sig_body(g, _):
            v = outb[pl.ds(g * 16, 16)] + bias
            outb[pl.ds(g * 16, 16)] = 1.0 / (1.0 + jnp.exp(-v))
            return 0

        lax.fori_loop(0, GROUPS, sig_body, 0)
        pltpu.sync_copy(outb.at[pl.ds(0, CHUNK)],
                        out_hbm.at[pl.ds(out_base + c * CHUNK, CHUNK)])

    start_gathers(0, 0)

    def pair_body(m, _):
        c0 = m * 2
        start_gathers(c0 + 1, 1)
        wait_gathers(0)
        compute_chunk(c0, 0)

        @pl.when(c0 + 2 < N_CHUNKS)
        def _():
            start_gathers(c0 + 2, 0)

        wait_gathers(1)
        compute_chunk(c0 + 1, 1)
        return 0

    lax.fori_loop(0, N_CHUNKS // 2, pair_body, 0)


@jax.jit
def _run(wi2d, ci2d, table, params):
    mesh = plsc.VectorSubcoreMesh(core_axis_name="c", subcore_axis_name="s")
    kern = pl.kernel(
        _sc_body,
        out_type=jax.ShapeDtypeStruct((TOTAL,), jnp.float32),
        mesh=mesh,
        scratch_types=[
            pltpu.VMEM((2, ROWS_PER_CHUNK, IDXW), jnp.int32),
            pltpu.VMEM((2, ROWS_PER_CHUNK, IDXW), jnp.int32),
            pltpu.VMEM((2, CHUNK, EMB_DIM), jnp.float32),
            pltpu.VMEM((2, CHUNK, EMB_DIM), jnp.float32),
            pltpu.VMEM((CHUNK + 16,), jnp.float32),
            pltpu.VMEM((EMB_DIM + 16,), jnp.float32),
            pltpu.SemaphoreType.DMA((2,)),
        ],
        compiler_params=pltpu.CompilerParams(
            needs_layout_passes=False, use_tc_tiling_on_sc=False),
    )
    return kern(wi2d, ci2d, table, params)


def kernel(word_ids, context_ids, table, fc_w, fc_b):
    wi2d = word_ids.reshape(TOTAL // IDXW, IDXW).astype(jnp.int32)
    ci2d = context_ids.reshape(TOTAL // IDXW, IDXW).astype(jnp.int32)
    params = jnp.concatenate(
        [fc_w.reshape(EMB_DIM), jnp.broadcast_to(fc_b, (16,))]).astype(jnp.float32)
    out = _run(wi2d, ci2d, table.astype(jnp.float32), params)
    return out.reshape(B, L, 1)


# speedup vs baseline: 4.9280x; 1.2810x over previous
import jax
import jax.numpy as jnp
from jax import lax
from jax.experimental import pallas as pl
from jax.experimental.pallas import tpu as pltpu
from jax.experimental.pallas import tpu_sc as plsc

B = 16384
L = 50
EMB_DIM = 64
N_WORDS_TOTAL = 1000000
HALF = N_WORDS_TOTAL // 2
TOTAL = B * L
NW = 32
PER_W = TOTAL // NW
CHUNK = 256
IDXW = 128
ROWS_PER_CHUNK = CHUNK // IDXW
N_CHUNKS = PER_W // CHUNK
GROUPS = CHUNK // 16
UNROLL = 8

BK = 2048
NBLK = 245
SPLIT = NBLK * BK
NROWS_LIN = 2 * SPLIT
IN_BLOCKS = (N_WORDS_TOTAL + BK - 1) // BK - 1


def _tr_body(a_ref, b_ref, o_ref):
    ident = (lax.broadcasted_iota(jnp.int32, (EMB_DIM, EMB_DIM), 0)
             == lax.broadcasted_iota(jnp.int32, (EMB_DIM, EMB_DIM), 1)
             ).astype(jnp.float32)
    dn = (((0,), (0,)), ((), ()))
    at = lax.dot_general(a_ref[...], ident, dn,
                         preferred_element_type=jnp.float32)
    bt = lax.dot_general(b_ref[...], ident, dn,
                         preferred_element_type=jnp.float32)
    o_ref[...] = jnp.concatenate([at, bt], axis=1)


def _transpose_pack(tt):
    return pl.pallas_call(
        _tr_body,
        grid=(NBLK,),
        in_specs=[
            pl.BlockSpec((EMB_DIM, BK), lambda k: (0, k)),
            pl.BlockSpec((EMB_DIM, BK),
                         lambda k: (0, jnp.minimum(NBLK + k, IN_BLOCKS))),
        ],
        out_specs=pl.BlockSpec((BK, 2 * EMB_DIM), lambda k: (k, 0)),
        out_shape=jax.ShapeDtypeStruct((SPLIT, 2 * EMB_DIM), jnp.float32),
    )(tt, tt)


def _sc_body(wi_hbm, ci_hbm, table_hbm, params_hbm, out_hbm,
             wi_v, ci_v, wr, cr, outb, pv, sems):
    nc = 2
    wid = lax.axis_index("s") * nc + lax.axis_index("c")

    pltpu.sync_copy(params_hbm, pv)
    bias = pv[pl.ds(EMB_DIM, 16)]
    fw = [pv[pl.ds(k * 16, 16)] for k in range(EMB_DIM // 16)]
    mask15 = lax.iota(jnp.int32, 16) == 15

    idx_row0 = wid * (PER_W // IDXW)
    out_base = wid * PER_W

    def start_gathers(c, buf):
        row = idx_row0 + c * ROWS_PER_CHUNK
        pltpu.sync_copy(wi_hbm.at[pl.ds(row, ROWS_PER_CHUNK)], wi_v.at[buf])
        pltpu.sync_copy(ci_hbm.at[pl.ds(row, ROWS_PER_CHUNK)], ci_v.at[buf])
        for j in range(ROWS_PER_CHUNK):
            pltpu.async_copy(table_hbm.at[wi_v.at[buf, j]],
                             wr.at[buf, pl.ds(j * IDXW, IDXW)], sems.at[buf])
            pltpu.async_copy(table_hbm.at[ci_v.at[buf, j]],
                             cr.at[buf, pl.ds(j * IDXW, IDXW)], sems.at[buf])

    def wait_gathers(buf):
        for j in range(ROWS_PER_CHUNK):
            pltpu.make_async_copy(table_hbm.at[wi_v.at[buf, j]],
                                  wr.at[buf, pl.ds(j * IDXW, IDXW)],
                                  sems.at[buf]).wait()
            pltpu.make_async_copy(table_hbm.at[ci_v.at[buf, j]],
                                  cr.at[buf, pl.ds(j * IDXW, IDXW)],
                                  sems.at[buf]).wait()

    def compute_chunk(c, buf):
        def look_body(i, _):
            accs = []
            for u in range(UNROLL):
                ii = i * UNROLL + u
                acc = None
                for k in range(EMB_DIM // 16):
                    t = (wr[buf, ii, pl.ds(k * 16, 16)]
                         * cr[buf, ii, pl.ds(k * 16, 16)]) * fw[k]
                    acc = t if acc is None else acc + t
                accs.append(acc)
            cums = [plsc.cumsum(a) for a in accs]
            for u, cum in enumerate(cums):
                plsc.store_compressed(outb.at[pl.ds(i * UNROLL + u, 16)],
                                      cum, mask=mask15)
            return 0

        lax.fori_loop(0, CHUNK // UNROLL, look_body, 0)

        def sig_body(g, _):
            v = outb[pl.ds(g * 16, 16)] + bias
            outb[pl.ds(g * 16, 16)] = 1.0 / (1.0 + jnp.exp(-v))
            return 0

        lax.fori_loop(0, GROUPS, sig_body, 0)
        pltpu.sync_copy(outb.at[pl.ds(0, CHUNK)],
                        out_hbm.at[pl.ds(out_base + c * CHUNK, CHUNK)])

    start_gathers(0, 0)

    def pair_body(m, _):
        c0 = m * 2
        start_gathers(c0 + 1, 1)
        wait_gathers(0)
        compute_chunk(c0, 0)

        @pl.when(c0 + 2 < N_CHUNKS)
        def _():
            start_gathers(c0 + 2, 0)

        wait_gathers(1)
        compute_chunk(c0 + 1, 1)
        return 0

    lax.fori_loop(0, N_CHUNKS // 2, pair_body, 0)


@jax.jit
def _run(wi2d, ci2d, table, params):
    mesh = plsc.VectorSubcoreMesh(core_axis_name="c", subcore_axis_name="s")
    kern = pl.kernel(
        _sc_body,
        out_type=jax.ShapeDtypeStruct((TOTAL,), jnp.float32),
        mesh=mesh,
        scratch_types=[
            pltpu.VMEM((2, ROWS_PER_CHUNK, IDXW), jnp.int32),
            pltpu.VMEM((2, ROWS_PER_CHUNK, IDXW), jnp.int32),
            pltpu.VMEM((2, CHUNK, EMB_DIM), jnp.float32),
            pltpu.VMEM((2, CHUNK, EMB_DIM), jnp.float32),
            pltpu.VMEM((CHUNK + 16,), jnp.float32),
            pltpu.VMEM((EMB_DIM + 16,), jnp.float32),
            pltpu.SemaphoreType.DMA((2,)),
        ],
        compiler_params=pltpu.CompilerParams(
            needs_layout_passes=False, use_tc_tiling_on_sc=False),
    )
    return kern(wi2d, ci2d, table, params)


def kernel(word_ids, context_ids, table, fc_w, fc_b):
    packed = _transpose_pack(table.T.astype(jnp.float32))
    table_lin = packed.reshape(NROWS_LIN, EMB_DIM)

    def remap(ids):
        i = ids.reshape(TOTAL // IDXW, IDXW).astype(jnp.int32)
        return jnp.where(i < SPLIT, 2 * i, 2 * i - (NROWS_LIN - 1))

    wi2d = remap(word_ids)
    ci2d = remap(context_ids)
    params = jnp.concatenate(
        [fc_w.reshape(EMB_DIM), jnp.broadcast_to(fc_b, (16,))]).astype(jnp.float32)
    out = _run(wi2d, ci2d, table_lin, params)
    return out.reshape(B, L, 1)
